# baseline (speedup 1.0000x reference)
import functools

import jax
import jax.numpy as jnp
from jax.experimental import pallas as pl
from jax.experimental.pallas import tpu as pltpu

_BV = 4096
_CV = 1024
_CR = 8


def _rotl(x, d):
    return jnp.left_shift(x, jnp.uint32(d)) | jnp.right_shift(x, jnp.uint32(32 - d))


def _threefry_bits(k0, k1, x1_init, shape):
    ks2 = k0 ^ k1 ^ jnp.uint32(0x1BD11BDA)
    x0 = jnp.broadcast_to(k0, shape)
    x1 = jnp.broadcast_to(x1_init, shape)
    rots = ((13, 15, 26, 6), (17, 29, 16, 24))
    ksv = (k0, k1, ks2)
    inj1 = tuple(ksv[(r + 1) % 3] for r in range(5))
    inj2 = tuple(ksv[(r + 2) % 3] + jnp.uint32(r + 1) for r in range(5))
    for r in range(5):
        for d in rots[r % 2]:
            x0 = x0 + x1
            x1 = _rotl(x1, d)
            x1 = x1 ^ x0
        x0 = x0 + inj1[r]
        x1 = x1 + inj2[r]
    return x0 ^ x1


def _body(logits_ref, k0_ref, k1_ref, st_ref, nz_ref, out_ref, bv_ref, bi_ref,
          *, nv, vocab, rows):
    v = pl.program_id(0)

    @pl.when(v == 0)
    def _():
        bv_ref[...] = jnp.full((rows, _BV), -jnp.inf, jnp.float32)
        bi_ref[...] = jnp.full((rows, _BV), jnp.int32(2147483647), jnp.int32)

    for r in range(rows // _CR):
        rs = pl.ds(r * _CR, _CR)
        k0 = k0_ref[rs, :]
        k1 = k1_ref[rs, :]
        st = st_ref[rs, :]
        nz = nz_ref[rs, :]
        for c in range(_BV // _CV):
            cols = (jax.lax.broadcasted_iota(jnp.int32, (1, _CV), 1)
                    + (v * _BV + c * _CV))
            bits = _threefry_bits(k0, k1, cols.astype(jnp.uint32) + k1,
                                  (_CR, _CV))
            mant = jnp.right_shift(bits, jnp.uint32(9)) | jnp.uint32(0x3F800000)
            u = jax.lax.bitcast_convert_type(mant, jnp.float32) - jnp.float32(1.0)
            g = -jnp.log(u + jnp.float32(1e-20))
            noise = -jnp.log(g + jnp.float32(1e-20))

            scaled = logits_ref[rs, pl.ds(c * _CV, _CV)] / st
            pert = scaled + noise * nz
            pert = jnp.where(cols < vocab, pert, -jnp.inf)

            cs = pl.ds(c * _CV, _CV)
            bv = bv_ref[rs, cs]
            take = pert > bv
            bv_ref[rs, cs] = jnp.where(take, pert, bv)
            bi_ref[rs, cs] = jnp.where(take, jnp.broadcast_to(cols, (_CR, _CV)),
                                       bi_ref[rs, cs])

    @pl.when(v == nv - 1)
    def _():
        bv = bv_ref[...]
        m = jnp.max(bv, axis=1, keepdims=True)
        idx = jnp.min(jnp.where(bv == m, bi_ref[...], jnp.int32(2147483647)),
                      axis=1, keepdims=True)
        out_ref[...] = idx


def kernel(logits, temperature, seed, pos, apply_temperature):
    rows, vocab = logits.shape
    logits = logits.astype(jnp.float32)

    kd = jax.vmap(
        lambda s, p: jax.random.key_data(jax.random.fold_in(jax.random.key(s), p))
    )(seed, pos)
    k0 = kd[:, 0:1]
    k1 = kd[:, 1:2]

    at = jnp.asarray(apply_temperature)
    safe_t = jnp.where(temperature == 0.0, jnp.float32(1.0), temperature)
    st_eff = jnp.where(at != 0, safe_t, jnp.float32(1.0))[:, None]
    nz = (temperature != 0.0).astype(jnp.float32)[:, None]

    nv = pl.cdiv(vocab, _BV)
    out = pl.pallas_call(
        functools.partial(_body, nv=nv, vocab=vocab, rows=rows),
        grid=(nv,),
        in_specs=[
            pl.BlockSpec((rows, _BV), lambda v: (0, v)),
            pl.BlockSpec((rows, 1), lambda v: (0, 0)),
            pl.BlockSpec((rows, 1), lambda v: (0, 0)),
            pl.BlockSpec((rows, 1), lambda v: (0, 0)),
            pl.BlockSpec((rows, 1), lambda v: (0, 0)),
        ],
        out_specs=pl.BlockSpec((rows, 1), lambda v: (0, 0)),
        out_shape=jax.ShapeDtypeStruct((rows, 1), jnp.int32),
        scratch_shapes=[
            pltpu.VMEM((rows, _BV), jnp.float32),
            pltpu.VMEM((rows, _BV), jnp.int32),
        ],
    )(logits, k0, k1, st_eff, nz)
    return out[:, 0]


# speedup vs baseline: 1.1643x; 1.1643x over previous
import functools

import jax
import jax.numpy as jnp
from jax import lax
from jax.experimental import pallas as pl
from jax.experimental.pallas import tpu as pltpu
from jax.experimental.pallas import tpu_sc as plsc

_BV = 2048
_CV = 1024
_CR = 8

_S = 32768
_SC_U = 8

_IMAX = 2147483647


def _rotl(x, d):
    return jnp.left_shift(x, jnp.uint32(d)) | jnp.right_shift(x, jnp.uint32(32 - d))


def _threefry_bits(k0, k1, x1_init, shape):
    ks2 = k0 ^ k1 ^ jnp.uint32(0x1BD11BDA)
    x0 = jnp.broadcast_to(k0, shape)
    x1 = jnp.broadcast_to(x1_init, shape)
    rots = ((13, 15, 26, 6), (17, 29, 16, 24))
    ksv = (k0, k1, ks2)
    inj1 = tuple(ksv[(r + 1) % 3] for r in range(5))
    inj2 = tuple(ksv[(r + 2) % 3] + jnp.uint32(r + 1) for r in range(5))
    for r in range(5):
        for d in rots[r % 2]:
            x0 = x0 + x1
            x1 = _rotl(x1, d)
            x1 = x1 ^ x0
        x0 = x0 + inj1[r]
        x1 = x1 + inj2[r]
    return x0 ^ x1


def _gumbel_from_bits(bits):
    mant = jnp.right_shift(bits, jnp.uint32(9)) | jnp.uint32(0x3F800000)
    u = jax.lax.bitcast_convert_type(mant, jnp.float32) - jnp.float32(1.0)
    g = -jnp.log(u + jnp.float32(1e-20))
    return -jnp.log(g + jnp.float32(1e-20))


def _sc_bits_body(k0_hbm, k1_hbm, out_hbm, kv0_buf, kv1_buf, row_buf):
    nc = 2
    w = lax.axis_index("s") * nc + lax.axis_index("c")
    row0 = w * 4
    pltpu.sync_copy(k0_hbm.at[pl.ds(row0, 4)], kv0_buf)
    pltpu.sync_copy(k1_hbm.at[pl.ds(row0, 4)], kv1_buf)
    step = 16 * _SC_U
    for lr in range(4):
        kv0 = kv0_buf[lr, :]
        kv1 = kv1_buf[lr, :]

        def grp(g, _, kv0=kv0, kv1=kv1):
            base = g * step
            for uu in range(_SC_U):
                cols = lax.iota(jnp.int32, 16) + (base + uu * 16)
                x1 = cols.astype(jnp.uint32) + kv1
                row_buf[pl.ds(base + uu * 16, 16)] = _threefry_bits(
                    kv0, kv1, x1, (16,))
            return 0

        lax.fori_loop(0, _S // step, grp, 0)
        pltpu.sync_copy(row_buf, out_hbm.at[row0 + lr, :])


def _sc_bits(k0b, k1b):
    mesh = plsc.VectorSubcoreMesh(core_axis_name="c", subcore_axis_name="s")
    fn = functools.partial(
        pl.kernel,
        mesh=mesh,
        out_type=jax.ShapeDtypeStruct((128, _S), jnp.uint32),
        scratch_types=[
            pltpu.VMEM((4, 16), jnp.uint32),
            pltpu.VMEM((4, 16), jnp.uint32),
            pltpu.VMEM((_S,), jnp.uint32),
        ],
    )(_sc_bits_body)
    return fn(k0b, k1b)


def _tc_main_body(logits_ref, k0_ref, k1_ref, st_ref, nz_ref,
                  bvp_ref, bip_ref, bv_ref, bi_ref, *, nv, vocab, rows):
    v = pl.program_id(0)

    @pl.when(v == 0)
    def _():
        bv_ref[...] = jnp.full((rows, _BV), -jnp.inf, jnp.float32)
        bi_ref[...] = jnp.full((rows, _BV), _IMAX, jnp.int32)

    for r in range(rows // _CR):
        rs = pl.ds(r * _CR, _CR)
        k0 = k0_ref[rs, :]
        k1 = k1_ref[rs, :]
        st = st_ref[rs, :]
        nz = nz_ref[rs, :]
        for c in range(_BV // _CV):
            cols = (jax.lax.broadcasted_iota(jnp.int32, (1, _CV), 1)
                    + (_S + v * _BV + c * _CV))
            bits = _threefry_bits(k0, k1, cols.astype(jnp.uint32) + k1,
                                  (_CR, _CV))
            noise = _gumbel_from_bits(bits)
            scaled = logits_ref[rs, pl.ds(c * _CV, _CV)] / st
            pert = scaled + noise * nz
            pert = jnp.where(cols < vocab, pert, -jnp.inf)

            cs = pl.ds(c * _CV, _CV)
            bv = bv_ref[rs, cs]
            take = pert > bv
            bv_ref[rs, cs] = jnp.where(take, pert, bv)
            bi_ref[rs, cs] = jnp.where(take, jnp.broadcast_to(cols, (_CR, _CV)),
                                       bi_ref[rs, cs])

    @pl.when(v == nv - 1)
    def _():
        bv = bv_ref[...]
        m = jnp.max(bv, axis=1, keepdims=True)
        idx = jnp.min(jnp.where(bv == m, bi_ref[...], _IMAX),
                      axis=1, keepdims=True)
        bvp_ref[...] = m
        bip_ref[...] = idx


def _tc_tail_body(bits_ref, logits_ref, st_ref, nz_ref, bvp_ref, bip_ref,
                  out_ref, bv_ref, bi_ref, *, nt, rows):
    v = pl.program_id(0)

    @pl.when(v == 0)
    def _():
        bv_ref[...] = jnp.full((rows, _BV), -jnp.inf, jnp.float32)
        bi_ref[...] = jnp.full((rows, _BV), _IMAX, jnp.int32)

    for r in range(rows // _CR):
        rs = pl.ds(r * _CR, _CR)
        st = st_ref[rs, :]
        nz = nz_ref[rs, :]
        for c in range(_BV // _CV):
            cols = (jax.lax.broadcasted_iota(jnp.int32, (1, _CV), 1)
                    + (v * _BV + c * _CV))
            cs = pl.ds(c * _CV, _CV)
            noise = _gumbel_from_bits(bits_ref[rs, cs])
            scaled = logits_ref[rs, cs] / st
            pert = scaled + noise * nz

            bv = bv_ref[rs, cs]
            take = pert > bv
            bv_ref[rs, cs] = jnp.where(take, pert, bv)
            bi_ref[rs, cs] = jnp.where(take, jnp.broadcast_to(cols, (_CR, _CV)),
                                       bi_ref[rs, cs])

    @pl.when(v == nt - 1)
    def _():
        bv = bv_ref[...]
        m = jnp.max(bv, axis=1, keepdims=True)
        idx = jnp.min(jnp.where(bv == m, bi_ref[...], _IMAX),
                      axis=1, keepdims=True)
        bvp = bvp_ref[...]
        bip = bip_ref[...]
        take = (m > bvp) | ((m == bvp) & (idx < bip))
        out_ref[...] = jnp.where(take, idx, bip)


def kernel(logits, temperature, seed, pos, apply_temperature):
    rows, vocab = logits.shape
    logits = logits.astype(jnp.float32)

    kd = jax.vmap(
        lambda s, p: jax.random.key_data(jax.random.fold_in(jax.random.key(s), p))
    )(seed, pos)
    k0 = kd[:, 0:1]
    k1 = kd[:, 1:2]

    at = jnp.asarray(apply_temperature)
    safe_t = jnp.where(temperature == 0.0, jnp.float32(1.0), temperature)
    st_eff = jnp.where(at != 0, safe_t, jnp.float32(1.0))[:, None]
    nz = (temperature != 0.0).astype(jnp.float32)[:, None]

    k0b = jnp.broadcast_to(k0, (rows, 16))
    k1b = jnp.broadcast_to(k1, (rows, 16))
    bits = _sc_bits(k0b, k1b)

    row_spec = pl.BlockSpec((rows, 1), lambda v: (0, 0))

    nv = pl.cdiv(vocab - _S, _BV)
    off = _S // _BV
    bvp, bip = pl.pallas_call(
        functools.partial(_tc_main_body, nv=nv, vocab=vocab, rows=rows),
        grid=(nv,),
        in_specs=[
            pl.BlockSpec((rows, _BV), lambda v: (0, v + off)),
            row_spec, row_spec, row_spec, row_spec,
        ],
        out_specs=[row_spec, row_spec],
        out_shape=[
            jax.ShapeDtypeStruct((rows, 1), jnp.float32),
            jax.ShapeDtypeStruct((rows, 1), jnp.int32),
        ],
        scratch_shapes=[
            pltpu.VMEM((rows, _BV), jnp.float32),
            pltpu.VMEM((rows, _BV), jnp.int32),
        ],
    )(logits, k0, k1, st_eff, nz)

    nt = _S // _BV
    out = pl.pallas_call(
        functools.partial(_tc_tail_body, nt=nt, rows=rows),
        grid=(nt,),
        in_specs=[
            pl.BlockSpec((rows, _BV), lambda v: (0, v)),
            pl.BlockSpec((rows, _BV), lambda v: (0, v)),
            row_spec, row_spec, row_spec, row_spec,
        ],
        out_specs=row_spec,
        out_shape=jax.ShapeDtypeStruct((rows, 1), jnp.int32),
        scratch_shapes=[
            pltpu.VMEM((rows, _BV), jnp.float32),
            pltpu.VMEM((rows, _BV), jnp.int32),
        ],
    )(bits, logits, st_eff, nz, bvp, bip)
    return out[:, 0]
